# baseline (speedup 1.0000x reference)
import functools

import jax
import jax.numpy as jnp
from jax import lax
from jax.experimental import pallas as pl
from jax.experimental.pallas import tpu as pltpu
from jax.experimental.pallas import tpu_sc as plsc

NC = 2
NS = 16
NW = NC * NS
K_EDGE = 96
G_DEG = 6
G_BIG = 2
G_SMALL = 6


def _mesh():
  return plsc.VectorSubcoreMesh(
      core_axis_name="c", subcore_axis_name="s", num_cores=NC,
      num_subcores=NS)


def _deg_body(npt, n_pad, dst_i, ones, init, out, dst_v, ones_v, sem, acc):
  cid = lax.axis_index("c")
  sid = lax.axis_index("s")
  wid = sid * NC + cid
  rpt = n_pad // NS
  sl = pl.ds(sid * rpt, rpt)
  pltpu.sync_copy(dst_i.at[wid], dst_v)
  pltpu.sync_copy(ones, ones_v)
  pltpu.sync_copy(init.at[sl], acc.at[sl])
  plsc.subcore_barrier()

  @pl.loop(0, npt // G_DEG)
  def _(i):
    base = G_DEG * i
    cps = [
        pltpu.async_copy(ones_v, acc.at[dst_v.at[base + b]], sem, add=True)
        for b in range(G_DEG)
    ]
    for cp in cps:
      cp.wait()

  plsc.subcore_barrier()
  pltpu.sync_copy(acc.at[sl], out.at[cid].at[sl])


def _make_deg(npt, n_pad):
  return pl.kernel(
      functools.partial(_deg_body, npt, n_pad),
      out_type=jax.ShapeDtypeStruct((NC, n_pad, 16), jnp.float32),
      mesh=_mesh(),
      compiler_params=pltpu.CompilerParams(use_tc_tiling_on_sc=False),
      scratch_types=[
          pltpu.VMEM((npt, K_EDGE), jnp.int32),
          pltpu.VMEM((K_EDGE, 16), jnp.float32),
          pltpu.SemaphoreType.DMA,
          pltpu.VMEM_SHARED((n_pad, 16), jnp.float32),
      ],
  )


def _agg_body(npt, n_pad, d, g, table, src_i, dst_i, zeros, out, *scratch):
  src_v, dst_v = scratch[0], scratch[1]
  rows = list(scratch[2:2 + g])
  sem_g, sem_s = scratch[2 + g], scratch[3 + g]
  acc = scratch[4 + g]
  cid = lax.axis_index("c")
  sid = lax.axis_index("s")
  wid = sid * NC + cid
  rpt = n_pad // NS
  sl = pl.ds(sid * rpt, rpt)
  pltpu.sync_copy(src_i.at[wid], src_v)
  pltpu.sync_copy(dst_i.at[wid], dst_v)
  pltpu.sync_copy(zeros.at[sl], acc.at[sl])
  plsc.subcore_barrier()

  @pl.loop(0, npt // g)
  def _(i):
    base = g * i
    gathers = [
        pltpu.async_copy(table.at[src_v.at[base + b]], rows[b], sem_g)
        for b in range(g)
    ]
    scatters = []
    for b in range(g):
      gathers[b].wait()
      scatters.append(
          pltpu.async_copy(rows[b], acc.at[dst_v.at[base + b]], sem_s,
                           add=True))
    for cp in scatters:
      cp.wait()

  plsc.subcore_barrier()
  pltpu.sync_copy(acc.at[sl], out.at[cid].at[sl])


def _make_agg(npt, n_pad, d, g):
  return pl.kernel(
      functools.partial(_agg_body, npt, n_pad, d, g),
      out_type=jax.ShapeDtypeStruct((NC, n_pad, d), jnp.float32),
      mesh=_mesh(),
      compiler_params=pltpu.CompilerParams(use_tc_tiling_on_sc=False),
      scratch_types=[
          pltpu.VMEM((npt, K_EDGE), jnp.int32),
          pltpu.VMEM((npt, K_EDGE), jnp.int32),
      ] + [pltpu.VMEM((K_EDGE, d), jnp.float32) for _ in range(g)] + [
          pltpu.SemaphoreType.DMA,
          pltpu.SemaphoreType.DMA,
          pltpu.VMEM_SHARED((n_pad, d), jnp.float32),
      ],
  )


def _s_block(degp, n, r0):
  dsum = degp[0, :, 0:1] + degp[1, :, 0:1] - 1.0
  s = jnp.where(dsum > 0, lax.rsqrt(jnp.maximum(dsum, 1e-12)), 0.0)
  rows = r0 + lax.broadcasted_iota(jnp.int32, s.shape, 0)
  return jnp.where(rows < n, s, 0.0)


def _lin_first_body(n, r, x_ref, w_ref, degp_ref, o_ref):
  i = pl.program_id(0)
  s = _s_block(degp_ref[...], n, i * r)
  o_ref[...] = s * jnp.dot(x_ref[...], w_ref[...],
                           preferred_element_type=jnp.float32)


def _lin_mid_body(n, r, p_ref, t_ref, b_ref, w_ref, degp_ref, o_ref):
  i = pl.program_id(0)
  s = _s_block(degp_ref[...], n, i * r)
  z = s * (p_ref[0] + p_ref[1] + t_ref[...]) + b_ref[...]
  a = jnp.maximum(z, 0.0)
  o_ref[...] = s * jnp.dot(a, w_ref[...], preferred_element_type=jnp.float32)


def _final_body(n, r, nvalid, p_ref, t_ref, b_ref, degp_ref, o_ref):
  i = pl.program_id(0)
  s = _s_block(degp_ref[...], n, i * r)
  z = s * (p_ref[0] + p_ref[1] + t_ref[...]) + b_ref[...]
  col = lax.broadcasted_iota(jnp.int32, z.shape, 1)
  valid = col < nvalid
  zm = jnp.where(valid, z, -jnp.inf)
  m = jnp.max(zm, axis=1, keepdims=True)
  e = jnp.where(valid, jnp.exp(zm - m), 0.0)
  lse = jnp.log(jnp.sum(e, axis=1, keepdims=True))
  o_ref[...] = z - m - lse


_R = 512


def _tc_first(n, n_pad, din, dout):
  grid = n_pad // _R
  return pl.pallas_call(
      functools.partial(_lin_first_body, n, _R),
      grid=(grid,),
      in_specs=[
          pl.BlockSpec((_R, din), lambda i: (i, 0)),
          pl.BlockSpec((din, dout), lambda i: (0, 0)),
          pl.BlockSpec((NC, _R, 16), lambda i: (0, i, 0)),
      ],
      out_specs=pl.BlockSpec((_R, dout), lambda i: (i, 0)),
      out_shape=jax.ShapeDtypeStruct((n_pad, dout), jnp.float32),
  )


def _tc_mid(n, n_pad, din, dout):
  grid = n_pad // _R
  return pl.pallas_call(
      functools.partial(_lin_mid_body, n, _R),
      grid=(grid,),
      in_specs=[
          pl.BlockSpec((NC, _R, din), lambda i: (0, i, 0)),
          pl.BlockSpec((_R, din), lambda i: (i, 0)),
          pl.BlockSpec((1, din), lambda i: (0, 0)),
          pl.BlockSpec((din, dout), lambda i: (0, 0)),
          pl.BlockSpec((NC, _R, 16), lambda i: (0, i, 0)),
      ],
      out_specs=pl.BlockSpec((_R, dout), lambda i: (i, 0)),
      out_shape=jax.ShapeDtypeStruct((n_pad, dout), jnp.float32),
  )


def _tc_final(n, n_pad, d, nvalid):
  grid = n_pad // _R
  return pl.pallas_call(
      functools.partial(_final_body, n, _R, nvalid),
      grid=(grid,),
      in_specs=[
          pl.BlockSpec((NC, _R, d), lambda i: (0, i, 0)),
          pl.BlockSpec((_R, d), lambda i: (i, 0)),
          pl.BlockSpec((1, d), lambda i: (0, 0)),
          pl.BlockSpec((NC, _R, 16), lambda i: (0, i, 0)),
      ],
      out_specs=pl.BlockSpec((_R, d), lambda i: (i, 0)),
      out_shape=jax.ShapeDtypeStruct((n_pad, d), jnp.float32),
  )


def kernel(x, edge_index, W1, b1, W2, b2, W3, b3):
  n, in_dim = x.shape
  e = edge_index.shape[1]
  h1 = W1.shape[1]
  h2 = W2.shape[1]
  dout = W3.shape[1]
  dout_p = ((dout + 15) // 16) * 16

  n_pad = ((n + _R - 1) // _R) * _R
  npt = (e + NW * K_EDGE - 1) // (NW * K_EDGE)
  npt = ((npt + G_DEG - 1) // G_DEG) * G_DEG
  e_pad = NW * npt * K_EDGE

  src = edge_index[0].astype(jnp.int32)
  dst = edge_index[1].astype(jnp.int32)
  pad = jnp.full((e_pad - e,), n, dtype=jnp.int32)
  src_i = jnp.concatenate([src, pad]).reshape(NW, npt, K_EDGE)
  dst_i = jnp.concatenate([dst, pad]).reshape(NW, npt, K_EDGE)

  x_pad = jnp.pad(x, ((0, n_pad - n), (0, 0)))
  w3p = jnp.pad(W3, ((0, 0), (0, dout_p - dout)))
  b1r = b1.reshape(1, h1)
  b2r = b2.reshape(1, h2)
  b3r = jnp.pad(b3, (0, dout_p - dout)).reshape(1, dout_p)

  ones16 = jnp.ones((K_EDGE, 16), jnp.float32)
  init16 = jnp.ones((n_pad, 16), jnp.float32)

  degp = _make_deg(npt, n_pad)(dst_i, ones16, init16)

  t1 = _tc_first(n, n_pad, in_dim, h1)(x_pad, W1, degp)
  t1, sa, da, za = lax.optimization_barrier(
      (t1, src_i, dst_i, jnp.zeros((n_pad, h1), jnp.float32)))
  p1 = _make_agg(npt, n_pad, h1, G_BIG)(t1, sa, da, za)
  t2 = _tc_mid(n, n_pad, h1, h2)(p1, t1, b1r, W2, degp)
  t2, sa, da, za = lax.optimization_barrier(
      (t2, src_i, dst_i, jnp.zeros((n_pad, h2), jnp.float32)))
  p2 = _make_agg(npt, n_pad, h2, G_SMALL)(t2, sa, da, za)
  t3 = _tc_mid(n, n_pad, h2, dout_p)(p2, t2, b2r, w3p, degp)
  t3, sa, da, za = lax.optimization_barrier(
      (t3, src_i, dst_i, jnp.zeros((n_pad, dout_p), jnp.float32)))
  p3 = _make_agg(npt, n_pad, dout_p, G_SMALL)(t3, sa, da, za)
  o = _tc_final(n, n_pad, dout_p, dout)(p3, t3, b3r, degp)
  return o[:n, :dout]


# speedup vs baseline: 1.4122x; 1.4122x over previous
import functools

import jax
import jax.numpy as jnp
from jax import lax
from jax.experimental import pallas as pl
from jax.experimental.pallas import tpu as pltpu
from jax.experimental.pallas import tpu_sc as plsc

NC = 2
NS = 16
NW = NC * NS
K_EDGE = 96


def _mesh():
  return plsc.VectorSubcoreMesh(
      core_axis_name="c", subcore_axis_name="s", num_cores=NC,
      num_subcores=NS)


def _deg_body(npt, n_pad, dst_i, ones, init, out, dst_v, ones_v, acc):
  cid = lax.axis_index("c")
  sid = lax.axis_index("s")
  wid = sid * NC + cid
  rpt = n_pad // NS
  sl = pl.ds(sid * rpt, rpt)
  pltpu.sync_copy(dst_i.at[wid], dst_v)
  pltpu.sync_copy(ones, ones_v)
  pltpu.sync_copy(init.at[sl], acc.at[sl])
  plsc.subcore_barrier()

  @pl.loop(0, npt)
  def _(j):
    pltpu.sync_copy(ones_v, acc.at[dst_v.at[j]], add=True)

  plsc.subcore_barrier()
  pltpu.sync_copy(acc.at[sl], out.at[cid].at[sl])


def _make_deg(npt, n_pad):
  return pl.kernel(
      functools.partial(_deg_body, npt, n_pad),
      out_type=jax.ShapeDtypeStruct((NC, n_pad, 16), jnp.float32),
      mesh=_mesh(),
      compiler_params=pltpu.CompilerParams(use_tc_tiling_on_sc=False),
      scratch_types=[
          pltpu.VMEM((npt, K_EDGE), jnp.int32),
          pltpu.VMEM((K_EDGE, 16), jnp.float32),
          pltpu.VMEM_SHARED((n_pad, 16), jnp.float32),
      ],
  )


def _agg_body(npt, n_pad, d, table, src_i, dst_i, zeros, out,
              src_v, dst_v, rows0, rows1, sem0, sem1, acc):
  cid = lax.axis_index("c")
  sid = lax.axis_index("s")
  wid = sid * NC + cid
  rpt = n_pad // NS
  sl = pl.ds(sid * rpt, rpt)
  rows = [rows0, rows1]
  sems = [sem0, sem1]
  pltpu.sync_copy(src_i.at[wid], src_v)
  pltpu.sync_copy(dst_i.at[wid], dst_v)
  pltpu.sync_copy(zeros.at[sl], acc.at[sl])
  plsc.subcore_barrier()

  pltpu.async_copy(table.at[src_v.at[0]], rows[0], sems[0])

  @pl.loop(0, npt // 2)
  def _(i):
    for b in range(2):
      j = 2 * i + b
      jnext = jnp.minimum(j + 1, npt - 1)
      pltpu.make_async_copy(table.at[src_v.at[j]], rows[b], sems[b]).wait()
      pltpu.async_copy(table.at[src_v.at[jnext]], rows[1 - b], sems[1 - b])
      pltpu.sync_copy(rows[b], acc.at[dst_v.at[j]], add=True)

  pltpu.make_async_copy(table.at[src_v.at[0]], rows[0], sems[0]).wait()
  plsc.subcore_barrier()
  pltpu.sync_copy(acc.at[sl], out.at[cid].at[sl])


def _make_agg(npt, n_pad, d):
  return pl.kernel(
      functools.partial(_agg_body, npt, n_pad, d),
      out_type=jax.ShapeDtypeStruct((NC, n_pad, d), jnp.float32),
      mesh=_mesh(),
      compiler_params=pltpu.CompilerParams(use_tc_tiling_on_sc=False),
      scratch_types=[
          pltpu.VMEM((npt, K_EDGE), jnp.int32),
          pltpu.VMEM((npt, K_EDGE), jnp.int32),
          pltpu.VMEM((K_EDGE, d), jnp.float32),
          pltpu.VMEM((K_EDGE, d), jnp.float32),
          pltpu.SemaphoreType.DMA,
          pltpu.SemaphoreType.DMA,
          pltpu.VMEM_SHARED((n_pad, d), jnp.float32),
      ],
  )


def _s_block(degp, n, r0):
  dsum = degp[0, :, 0:1] + degp[1, :, 0:1] - 1.0
  s = jnp.where(dsum > 0, lax.rsqrt(jnp.maximum(dsum, 1e-12)), 0.0)
  rows = r0 + lax.broadcasted_iota(jnp.int32, s.shape, 0)
  return jnp.where(rows < n, s, 0.0)


def _lin_first_body(n, r, x_ref, w_ref, degp_ref, o_ref):
  i = pl.program_id(0)
  s = _s_block(degp_ref[...], n, i * r)
  o_ref[...] = s * jnp.dot(x_ref[...], w_ref[...],
                           preferred_element_type=jnp.float32)


def _lin_mid_body(n, r, p_ref, t_ref, b_ref, w_ref, degp_ref, o_ref):
  i = pl.program_id(0)
  s = _s_block(degp_ref[...], n, i * r)
  z = s * (p_ref[0] + p_ref[1] + t_ref[...]) + b_ref[...]
  a = jnp.maximum(z, 0.0)
  o_ref[...] = s * jnp.dot(a, w_ref[...], preferred_element_type=jnp.float32)


def _final_body(n, r, nvalid, p_ref, t_ref, b_ref, degp_ref, o_ref):
  i = pl.program_id(0)
  s = _s_block(degp_ref[...], n, i * r)
  z = s * (p_ref[0] + p_ref[1] + t_ref[...]) + b_ref[...]
  col = lax.broadcasted_iota(jnp.int32, z.shape, 1)
  valid = col < nvalid
  zm = jnp.where(valid, z, -jnp.inf)
  m = jnp.max(zm, axis=1, keepdims=True)
  e = jnp.where(valid, jnp.exp(zm - m), 0.0)
  lse = jnp.log(jnp.sum(e, axis=1, keepdims=True))
  o_ref[...] = z - m - lse


_R = 512


def _tc_first(n, n_pad, din, dout):
  grid = n_pad // _R
  return pl.pallas_call(
      functools.partial(_lin_first_body, n, _R),
      grid=(grid,),
      in_specs=[
          pl.BlockSpec((_R, din), lambda i: (i, 0)),
          pl.BlockSpec((din, dout), lambda i: (0, 0)),
          pl.BlockSpec((NC, _R, 16), lambda i: (0, i, 0)),
      ],
      out_specs=pl.BlockSpec((_R, dout), lambda i: (i, 0)),
      out_shape=jax.ShapeDtypeStruct((n_pad, dout), jnp.float32),
  )


def _tc_mid(n, n_pad, din, dout):
  grid = n_pad // _R
  return pl.pallas_call(
      functools.partial(_lin_mid_body, n, _R),
      grid=(grid,),
      in_specs=[
          pl.BlockSpec((NC, _R, din), lambda i: (0, i, 0)),
          pl.BlockSpec((_R, din), lambda i: (i, 0)),
          pl.BlockSpec((1, din), lambda i: (0, 0)),
          pl.BlockSpec((din, dout), lambda i: (0, 0)),
          pl.BlockSpec((NC, _R, 16), lambda i: (0, i, 0)),
      ],
      out_specs=pl.BlockSpec((_R, dout), lambda i: (i, 0)),
      out_shape=jax.ShapeDtypeStruct((n_pad, dout), jnp.float32),
  )


def _tc_final(n, n_pad, d, nvalid):
  grid = n_pad // _R
  return pl.pallas_call(
      functools.partial(_final_body, n, _R, nvalid),
      grid=(grid,),
      in_specs=[
          pl.BlockSpec((NC, _R, d), lambda i: (0, i, 0)),
          pl.BlockSpec((_R, d), lambda i: (i, 0)),
          pl.BlockSpec((1, d), lambda i: (0, 0)),
          pl.BlockSpec((NC, _R, 16), lambda i: (0, i, 0)),
      ],
      out_specs=pl.BlockSpec((_R, d), lambda i: (i, 0)),
      out_shape=jax.ShapeDtypeStruct((n_pad, d), jnp.float32),
  )


def kernel(x, edge_index, W1, b1, W2, b2, W3, b3):
  n, in_dim = x.shape
  e = edge_index.shape[1]
  h1 = W1.shape[1]
  h2 = W2.shape[1]
  dout = W3.shape[1]
  dout_p = ((dout + 15) // 16) * 16

  n_pad = ((n + _R - 1) // _R) * _R
  npt = (e + NW * K_EDGE - 1) // (NW * K_EDGE)
  npt = npt + (npt % 2)
  e_pad = NW * npt * K_EDGE

  src = edge_index[0].astype(jnp.int32)
  dst = edge_index[1].astype(jnp.int32)
  pad = jnp.full((e_pad - e,), n, dtype=jnp.int32)
  src_i = jnp.concatenate([src, pad]).reshape(NW, npt, K_EDGE)
  dst_i = jnp.concatenate([dst, pad]).reshape(NW, npt, K_EDGE)

  x_pad = jnp.pad(x, ((0, n_pad - n), (0, 0)))
  w3p = jnp.pad(W3, ((0, 0), (0, dout_p - dout)))
  b1r = b1.reshape(1, h1)
  b2r = b2.reshape(1, h2)
  b3r = jnp.pad(b3, (0, dout_p - dout)).reshape(1, dout_p)

  ones16 = jnp.ones((K_EDGE, 16), jnp.float32)
  init16 = jnp.ones((n_pad, 16), jnp.float32)

  degp = _make_deg(npt, n_pad)(dst_i, ones16, init16)

  t1 = _tc_first(n, n_pad, in_dim, h1)(x_pad, W1, degp)
  t1, sa, da, za = lax.optimization_barrier(
      (t1, src_i, dst_i, jnp.zeros((n_pad, h1), jnp.float32)))
  p1 = _make_agg(npt, n_pad, h1)(t1, sa, da, za)
  t2 = _tc_mid(n, n_pad, h1, h2)(p1, t1, b1r, W2, degp)
  t2, sa, da, za = lax.optimization_barrier(
      (t2, src_i, dst_i, jnp.zeros((n_pad, h2), jnp.float32)))
  p2 = _make_agg(npt, n_pad, h2)(t2, sa, da, za)
  t3 = _tc_mid(n, n_pad, h2, dout_p)(p2, t2, b2r, w3p, degp)
  t3, sa, da, za = lax.optimization_barrier(
      (t3, src_i, dst_i, jnp.zeros((n_pad, dout_p), jnp.float32)))
  p3 = _make_agg(npt, n_pad, dout_p)(t3, sa, da, za)
  o = _tc_final(n, n_pad, dout_p, dout)(p3, t3, b3r, degp)
  return o[:n, :dout]


# speedup vs baseline: 2.0483x; 1.4504x over previous
import functools

import jax
import jax.numpy as jnp
from jax import lax
from jax.experimental import pallas as pl
from jax.experimental.pallas import tpu as pltpu
from jax.experimental.pallas import tpu_sc as plsc

NC = 2
NS = 16
NW = NC * NS
K_EDGE = 96


def _mesh():
  return plsc.VectorSubcoreMesh(
      core_axis_name="c", subcore_axis_name="s", num_cores=NC,
      num_subcores=NS)


def _deg_body(npt, n_pad, dst_i, ones, zeros16, out, dst_v, ones_v, acc):
  cid = lax.axis_index("c")
  sid = lax.axis_index("s")
  wid = sid * NC + cid
  rpt = n_pad // NS
  sl = pl.ds(sid * rpt, rpt)
  pltpu.sync_copy(dst_i.at[wid], dst_v)
  pltpu.sync_copy(ones, ones_v)
  pltpu.sync_copy(zeros16.at[sl], acc.at[sl])
  plsc.subcore_barrier()

  @pl.loop(0, npt)
  def _(j):
    pltpu.sync_copy(ones_v, acc.at[dst_v.at[j]], add=True)

  plsc.subcore_barrier()
  pltpu.sync_copy(acc.at[sl], out.at[cid].at[sl])


def _make_deg(npt, n_pad):
  return pl.kernel(
      functools.partial(_deg_body, npt, n_pad),
      out_type=jax.ShapeDtypeStruct((NC, n_pad, 16), jnp.float32),
      mesh=_mesh(),
      compiler_params=pltpu.CompilerParams(use_tc_tiling_on_sc=False),
      scratch_types=[
          pltpu.VMEM((npt, K_EDGE), jnp.int32),
          pltpu.VMEM((K_EDGE, 16), jnp.float32),
          pltpu.VMEM_SHARED((n_pad, 16), jnp.float32),
      ],
  )


def _agg_body(npt, n_pad, d, table, src_i, dst_i, zeros, out,
              src_v, dst_v, rows0, rows1, sem0, sem1, acc):
  cid = lax.axis_index("c")
  sid = lax.axis_index("s")
  wid = sid * NC + cid
  rpt = n_pad // NS
  sl = pl.ds(sid * rpt, rpt)
  rows = [rows0, rows1]
  sems = [sem0, sem1]
  pltpu.sync_copy(src_i.at[wid], src_v)
  pltpu.sync_copy(dst_i.at[wid], dst_v)
  pltpu.sync_copy(zeros.at[sl], acc.at[sl])
  plsc.subcore_barrier()

  pltpu.async_copy(table.at[src_v.at[0]], rows[0], sems[0])

  @pl.loop(0, npt // 2)
  def _(i):
    for b in range(2):
      j = 2 * i + b
      jnext = jnp.minimum(j + 1, npt - 1)
      pltpu.make_async_copy(table.at[src_v.at[j]], rows[b], sems[b]).wait()
      pltpu.async_copy(table.at[src_v.at[jnext]], rows[1 - b], sems[1 - b])
      pltpu.sync_copy(rows[b], acc.at[dst_v.at[j]], add=True)

  pltpu.make_async_copy(table.at[src_v.at[0]], rows[0], sems[0]).wait()
  plsc.subcore_barrier()
  pltpu.sync_copy(acc.at[sl], out.at[cid].at[sl])


def _make_agg(npt, n_pad, d):
  return pl.kernel(
      functools.partial(_agg_body, npt, n_pad, d),
      out_type=jax.ShapeDtypeStruct((NC, n_pad, d), jnp.float32),
      mesh=_mesh(),
      compiler_params=pltpu.CompilerParams(use_tc_tiling_on_sc=False),
      scratch_types=[
          pltpu.VMEM((npt, K_EDGE), jnp.int32),
          pltpu.VMEM((npt, K_EDGE), jnp.int32),
          pltpu.VMEM((K_EDGE, d), jnp.float32),
          pltpu.VMEM((K_EDGE, d), jnp.float32),
          pltpu.SemaphoreType.DMA,
          pltpu.SemaphoreType.DMA,
          pltpu.VMEM_SHARED((n_pad, d), jnp.float32),
      ],
  )


def _s_block(degp, n, r0):
  dsum = degp[0, :, 0:1] + degp[1, :, 0:1]
  s = jnp.where(dsum > 0, lax.rsqrt(jnp.maximum(dsum, 1e-12)), 0.0)
  rows = r0 + lax.broadcasted_iota(jnp.int32, s.shape, 0)
  return jnp.where(rows < n, s, 0.0)


def _lin_first_body(n, r, x_ref, w_ref, degp_ref, o_ref):
  i = pl.program_id(0)
  s = _s_block(degp_ref[...], n, i * r)
  o_ref[...] = s * jnp.dot(x_ref[...], w_ref[...],
                           preferred_element_type=jnp.float32)


def _lin_mid_body(n, r, p_ref, b_ref, w_ref, degp_ref, o_ref):
  i = pl.program_id(0)
  s = _s_block(degp_ref[...], n, i * r)
  z = s * (p_ref[0] + p_ref[1]) + b_ref[...]
  a = jnp.maximum(z, 0.0)
  o_ref[...] = s * jnp.dot(a, w_ref[...], preferred_element_type=jnp.float32)


def _final_body(n, r, nvalid, p_ref, b_ref, degp_ref, o_ref):
  i = pl.program_id(0)
  s = _s_block(degp_ref[...], n, i * r)
  z = s * (p_ref[0] + p_ref[1]) + b_ref[...]
  col = lax.broadcasted_iota(jnp.int32, z.shape, 1)
  valid = col < nvalid
  zm = jnp.where(valid, z, -jnp.inf)
  m = jnp.max(zm, axis=1, keepdims=True)
  e = jnp.where(valid, jnp.exp(zm - m), 0.0)
  lse = jnp.log(jnp.sum(e, axis=1, keepdims=True))
  o_ref[...] = z - m - lse


_R = 512


def _tc_first(n, n_pad, din, dout):
  grid = n_pad // _R
  return pl.pallas_call(
      functools.partial(_lin_first_body, n, _R),
      grid=(grid,),
      in_specs=[
          pl.BlockSpec((_R, din), lambda i: (i, 0)),
          pl.BlockSpec((din, dout), lambda i: (0, 0)),
          pl.BlockSpec((NC, _R, 16), lambda i: (0, i, 0)),
      ],
      out_specs=pl.BlockSpec((_R, dout), lambda i: (i, 0)),
      out_shape=jax.ShapeDtypeStruct((n_pad, dout), jnp.float32),
  )


def _tc_mid(n, n_pad, din, dout):
  grid = n_pad // _R
  return pl.pallas_call(
      functools.partial(_lin_mid_body, n, _R),
      grid=(grid,),
      in_specs=[
          pl.BlockSpec((NC, _R, din), lambda i: (0, i, 0)),
          pl.BlockSpec((1, din), lambda i: (0, 0)),
          pl.BlockSpec((din, dout), lambda i: (0, 0)),
          pl.BlockSpec((NC, _R, 16), lambda i: (0, i, 0)),
      ],
      out_specs=pl.BlockSpec((_R, dout), lambda i: (i, 0)),
      out_shape=jax.ShapeDtypeStruct((n_pad, dout), jnp.float32),
  )


def _tc_final(n, n_pad, d, nvalid):
  grid = n_pad // _R
  return pl.pallas_call(
      functools.partial(_final_body, n, _R, nvalid),
      grid=(grid,),
      in_specs=[
          pl.BlockSpec((NC, _R, d), lambda i: (0, i, 0)),
          pl.BlockSpec((1, d), lambda i: (0, 0)),
          pl.BlockSpec((NC, _R, 16), lambda i: (0, i, 0)),
      ],
      out_specs=pl.BlockSpec((_R, d), lambda i: (i, 0)),
      out_shape=jax.ShapeDtypeStruct((n_pad, d), jnp.float32),
  )


def kernel(x, edge_index, W1, b1, W2, b2, W3, b3):
  n, in_dim = x.shape
  e = edge_index.shape[1]
  h1 = W1.shape[1]
  h2 = W2.shape[1]
  dout = W3.shape[1]
  dout_p = ((dout + 15) // 16) * 16

  n_pad = ((n + _R - 1) // _R) * _R
  e_tot = e + n
  npt = (e_tot + NW * K_EDGE - 1) // (NW * K_EDGE)
  npt = npt + (npt % 2)
  e_pad = NW * npt * K_EDGE

  loop = jnp.arange(n, dtype=jnp.int32)
  src = jnp.concatenate([edge_index[0].astype(jnp.int32), loop])
  dst = jnp.concatenate([edge_index[1].astype(jnp.int32), loop])
  pad = jnp.full((e_pad - e_tot,), n, dtype=jnp.int32)
  src_i = jnp.concatenate([src, pad]).reshape(NW, npt, K_EDGE)
  dst_i = jnp.concatenate([dst, pad]).reshape(NW, npt, K_EDGE)

  x_pad = jnp.pad(x, ((0, n_pad - n), (0, 0)))
  w3p = jnp.pad(W3, ((0, 0), (0, dout_p - dout)))
  b1r = b1.reshape(1, h1)
  b2r = b2.reshape(1, h2)
  b3r = jnp.pad(b3, (0, dout_p - dout)).reshape(1, dout_p)

  ones16 = jnp.ones((K_EDGE, 16), jnp.float32)
  zeros16 = jnp.zeros((n_pad, 16), jnp.float32)

  degp = _make_deg(npt, n_pad)(dst_i, ones16, zeros16)

  t1 = _tc_first(n, n_pad, in_dim, h1)(x_pad, W1, degp)
  t1, sa, da, za = lax.optimization_barrier(
      (t1, src_i, dst_i, jnp.zeros((n_pad, h1), jnp.float32)))
  p1 = _make_agg(npt, n_pad, h1)(t1, sa, da, za)
  t2 = _tc_mid(n, n_pad, h1, h2)(p1, b1r, W2, degp)
  t2, sa, da, za = lax.optimization_barrier(
      (t2, src_i, dst_i, jnp.zeros((n_pad, h2), jnp.float32)))
  p2 = _make_agg(npt, n_pad, h2)(t2, sa, da, za)
  t3 = _tc_mid(n, n_pad, h2, dout_p)(p2, b2r, w3p, degp)
  t3, sa, da, za = lax.optimization_barrier(
      (t3, src_i, dst_i, jnp.zeros((n_pad, dout_p), jnp.float32)))
  p3 = _make_agg(npt, n_pad, dout_p)(t3, sa, da, za)
  o = _tc_final(n, n_pad, dout_p, dout)(p3, b3r, degp)
  return o[:n, :dout]


# speedup vs baseline: 2.1123x; 1.0312x over previous
import functools

import jax
import jax.numpy as jnp
from jax import lax
from jax.experimental import pallas as pl
from jax.experimental.pallas import tpu as pltpu
from jax.experimental.pallas import tpu_sc as plsc

NC = 2
NS = 16
NW = NC * NS
K_EDGE = 96


def _mesh():
  return plsc.VectorSubcoreMesh(
      core_axis_name="c", subcore_axis_name="s", num_cores=NC,
      num_subcores=NS)


def _deg_body(npt0, npt1, n_pad, dst_i, ones, zeros16, out, dst_v, ones_v,
              acc):
  cid = lax.axis_index("c")
  sid = lax.axis_index("s")
  wid = cid * NS + sid
  nloc = jnp.where(cid == 0, npt0, npt1)
  rpt = n_pad // NS
  sl = pl.ds(sid * rpt, rpt)
  pltpu.sync_copy(dst_i.at[wid], dst_v)
  pltpu.sync_copy(ones, ones_v)
  pltpu.sync_copy(zeros16.at[sl], acc.at[sl])
  plsc.subcore_barrier()

  @pl.loop(0, nloc)
  def _(j):
    pltpu.sync_copy(ones_v, acc.at[dst_v.at[j]], add=True)

  plsc.subcore_barrier()
  pltpu.sync_copy(acc.at[sl], out.at[cid].at[sl])


def _make_deg(npt0, npt1, n_pad):
  return pl.kernel(
      functools.partial(_deg_body, npt0, npt1, n_pad),
      out_type=jax.ShapeDtypeStruct((NC, n_pad, 16), jnp.float32),
      mesh=_mesh(),
      compiler_params=pltpu.CompilerParams(use_tc_tiling_on_sc=False),
      scratch_types=[
          pltpu.VMEM((npt0, K_EDGE), jnp.int32),
          pltpu.VMEM((K_EDGE, 16), jnp.float32),
          pltpu.VMEM_SHARED((n_pad, 16), jnp.float32),
      ],
  )


def _agg_body(npt0, npt1, n_pad, d, table, src_i, dst_i, zeros, out,
              src_v, dst_v, rows0, rows1, sem0, sem1, acc):
  cid = lax.axis_index("c")
  sid = lax.axis_index("s")
  wid = cid * NS + sid
  nloc = jnp.where(cid == 0, npt0, npt1)
  rpt = n_pad // NS
  sl = pl.ds(sid * rpt, rpt)
  rows = [rows0, rows1]
  sems = [sem0, sem1]
  pltpu.sync_copy(src_i.at[wid], src_v)
  pltpu.sync_copy(dst_i.at[wid], dst_v)
  pltpu.sync_copy(zeros.at[sl], acc.at[sl])
  plsc.subcore_barrier()

  pltpu.async_copy(table.at[src_v.at[0]], rows[0], sems[0])

  @pl.loop(0, nloc // 2)
  def _(i):
    for b in range(2):
      j = 2 * i + b
      jnext = jnp.minimum(j + 1, nloc - 1)
      pltpu.make_async_copy(table.at[src_v.at[j]], rows[b], sems[b]).wait()
      pltpu.async_copy(table.at[src_v.at[jnext]], rows[1 - b], sems[1 - b])
      pltpu.sync_copy(rows[b], acc.at[dst_v.at[j]], add=True)

  pltpu.make_async_copy(table.at[src_v.at[0]], rows[0], sems[0]).wait()
  plsc.subcore_barrier()
  pltpu.sync_copy(acc.at[sl], out.at[cid].at[sl])


def _make_agg(npt0, npt1, n_pad, d):
  return pl.kernel(
      functools.partial(_agg_body, npt0, npt1, n_pad, d),
      out_type=jax.ShapeDtypeStruct((NC, n_pad, d), jnp.float32),
      mesh=_mesh(),
      compiler_params=pltpu.CompilerParams(use_tc_tiling_on_sc=False),
      scratch_types=[
          pltpu.VMEM((npt0, K_EDGE), jnp.int32),
          pltpu.VMEM((npt0, K_EDGE), jnp.int32),
          pltpu.VMEM((K_EDGE, d), jnp.float32),
          pltpu.VMEM((K_EDGE, d), jnp.float32),
          pltpu.SemaphoreType.DMA,
          pltpu.SemaphoreType.DMA,
          pltpu.VMEM_SHARED((n_pad, d), jnp.float32),
      ],
  )


def _s_block(degp, n, r0):
  dsum = degp[0, :, 0:1] + degp[1, :, 0:1]
  s = jnp.where(dsum > 0, lax.rsqrt(jnp.maximum(dsum, 1e-12)), 0.0)
  rows = r0 + lax.broadcasted_iota(jnp.int32, s.shape, 0)
  return jnp.where(rows < n, s, 0.0)


def _lin_first_body(n, r, x_ref, w_ref, degp_ref, o_ref):
  i = pl.program_id(0)
  s = _s_block(degp_ref[...], n, i * r)
  o_ref[...] = s * jnp.dot(x_ref[...], w_ref[...],
                           preferred_element_type=jnp.float32)


def _lin_mid_body(n, r, p_ref, b_ref, w_ref, degp_ref, o_ref):
  i = pl.program_id(0)
  s = _s_block(degp_ref[...], n, i * r)
  z = s * (p_ref[0] + p_ref[1]) + b_ref[...]
  a = jnp.maximum(z, 0.0)
  o_ref[...] = s * jnp.dot(a, w_ref[...], preferred_element_type=jnp.float32)


def _final_body(n, r, nvalid, p_ref, b_ref, degp_ref, o_ref):
  i = pl.program_id(0)
  s = _s_block(degp_ref[...], n, i * r)
  z = s * (p_ref[0] + p_ref[1]) + b_ref[...]
  col = lax.broadcasted_iota(jnp.int32, z.shape, 1)
  valid = col < nvalid
  zm = jnp.where(valid, z, -jnp.inf)
  m = jnp.max(zm, axis=1, keepdims=True)
  e = jnp.where(valid, jnp.exp(zm - m), 0.0)
  lse = jnp.log(jnp.sum(e, axis=1, keepdims=True))
  o_ref[...] = z - m - lse


_R = 512


def _tc_first(n, n_pad, din, dout):
  grid = n_pad // _R
  return pl.pallas_call(
      functools.partial(_lin_first_body, n, _R),
      grid=(grid,),
      in_specs=[
          pl.BlockSpec((_R, din), lambda i: (i, 0)),
          pl.BlockSpec((din, dout), lambda i: (0, 0)),
          pl.BlockSpec((NC, _R, 16), lambda i: (0, i, 0)),
      ],
      out_specs=pl.BlockSpec((_R, dout), lambda i: (i, 0)),
      out_shape=jax.ShapeDtypeStruct((n_pad, dout), jnp.float32),
  )


def _tc_mid(n, n_pad, din, dout):
  grid = n_pad // _R
  return pl.pallas_call(
      functools.partial(_lin_mid_body, n, _R),
      grid=(grid,),
      in_specs=[
          pl.BlockSpec((NC, _R, din), lambda i: (0, i, 0)),
          pl.BlockSpec((1, din), lambda i: (0, 0)),
          pl.BlockSpec((din, dout), lambda i: (0, 0)),
          pl.BlockSpec((NC, _R, 16), lambda i: (0, i, 0)),
      ],
      out_specs=pl.BlockSpec((_R, dout), lambda i: (i, 0)),
      out_shape=jax.ShapeDtypeStruct((n_pad, dout), jnp.float32),
  )


def _tc_final(n, n_pad, d, nvalid):
  grid = n_pad // _R
  return pl.pallas_call(
      functools.partial(_final_body, n, _R, nvalid),
      grid=(grid,),
      in_specs=[
          pl.BlockSpec((NC, _R, d), lambda i: (0, i, 0)),
          pl.BlockSpec((1, d), lambda i: (0, 0)),
          pl.BlockSpec((NC, _R, 16), lambda i: (0, i, 0)),
      ],
      out_specs=pl.BlockSpec((_R, d), lambda i: (i, 0)),
      out_shape=jax.ShapeDtypeStruct((n_pad, d), jnp.float32),
  )


def kernel(x, edge_index, W1, b1, W2, b2, W3, b3):
  n, in_dim = x.shape
  e = edge_index.shape[1]
  h1 = W1.shape[1]
  h2 = W2.shape[1]
  dout = W3.shape[1]
  dout_p = ((dout + 15) // 16) * 16

  n_pad = ((n + _R - 1) // _R) * _R
  e_tot = e + n
  tot = (e_tot + NS * K_EDGE - 1) // (NS * K_EDGE)
  npt0 = min((int(tot * 0.57) + 1) // 2 * 2, 124)
  npt1 = (tot - npt0 + 1) // 2 * 2
  e0 = NS * npt0 * K_EDGE
  e_pad = e0 + NS * npt1 * K_EDGE

  loop = jnp.arange(n, dtype=jnp.int32)
  pad = jnp.full((e_pad - e_tot,), n, dtype=jnp.int32)
  src = jnp.concatenate([edge_index[0].astype(jnp.int32), loop, pad])
  dst = jnp.concatenate([edge_index[1].astype(jnp.int32), loop, pad])

  def _split(a):
    c0 = a[:e0].reshape(NS, npt0, K_EDGE)
    c1 = a[e0:].reshape(NS, npt1, K_EDGE)
    c1 = jnp.pad(c1, ((0, 0), (0, npt0 - npt1), (0, 0)), constant_values=n)
    return jnp.concatenate([c0, c1], axis=0)

  src_i = _split(src)
  dst_i = _split(dst)

  x_pad = jnp.pad(x, ((0, n_pad - n), (0, 0)))
  w3p = jnp.pad(W3, ((0, 0), (0, dout_p - dout)))
  b1r = b1.reshape(1, h1)
  b2r = b2.reshape(1, h2)
  b3r = jnp.pad(b3, (0, dout_p - dout)).reshape(1, dout_p)

  ones16 = jnp.ones((K_EDGE, 16), jnp.float32)
  zeros16 = jnp.zeros((n_pad, 16), jnp.float32)

  degp = _make_deg(npt0, npt1, n_pad)(dst_i, ones16, zeros16)

  t1 = _tc_first(n, n_pad, in_dim, h1)(x_pad, W1, degp)
  t1, sa, da, za = lax.optimization_barrier(
      (t1, src_i, dst_i, jnp.zeros((n_pad, h1), jnp.float32)))
  p1 = _make_agg(npt0, npt1, n_pad, h1)(t1, sa, da, za)
  t2 = _tc_mid(n, n_pad, h1, h2)(p1, b1r, W2, degp)
  t2, sa, da, za = lax.optimization_barrier(
      (t2, src_i, dst_i, jnp.zeros((n_pad, h2), jnp.float32)))
  p2 = _make_agg(npt0, npt1, n_pad, h2)(t2, sa, da, za)
  t3 = _tc_mid(n, n_pad, h2, dout_p)(p2, b2r, w3p, degp)
  t3, sa, da, za = lax.optimization_barrier(
      (t3, src_i, dst_i, jnp.zeros((n_pad, dout_p), jnp.float32)))
  p3 = _make_agg(npt0, npt1, n_pad, dout_p)(t3, sa, da, za)
  o = _tc_final(n, n_pad, dout_p, dout)(p3, b3r, degp)
  return o[:n, :dout]


# speedup vs baseline: 2.3913x; 1.1321x over previous
import functools

import jax
import jax.numpy as jnp
from jax import lax
from jax.experimental import pallas as pl
from jax.experimental.pallas import tpu as pltpu
from jax.experimental.pallas import tpu_sc as plsc

NC = 2
NS = 16
NW = NC * NS
K_EDGE = 96


def _mesh():
  return plsc.VectorSubcoreMesh(
      core_axis_name="c", subcore_axis_name="s", num_cores=NC,
      num_subcores=NS)


def _deg_body(npt0, npt1, n_pad, dst_i, ones, zeros16, out, dst_v, ones_v,
              acc):
  cid = lax.axis_index("c")
  sid = lax.axis_index("s")
  wid = cid * NS + sid
  nloc = jnp.where(cid == 0, npt0, npt1)
  rpt = n_pad // NS
  sl = pl.ds(sid * rpt, rpt)
  pltpu.sync_copy(dst_i.at[wid], dst_v)
  pltpu.sync_copy(ones, ones_v)
  pltpu.sync_copy(zeros16.at[sl], acc.at[sl])
  plsc.subcore_barrier()

  @pl.loop(0, nloc)
  def _(j):
    pltpu.sync_copy(ones_v, acc.at[dst_v.at[j]], add=True)

  plsc.subcore_barrier()
  pltpu.sync_copy(acc.at[sl], out.at[cid].at[sl])


def _make_deg(npt0, npt1, n_pad):
  return pl.kernel(
      functools.partial(_deg_body, npt0, npt1, n_pad),
      out_type=jax.ShapeDtypeStruct((NC, n_pad, 16), jnp.float32),
      mesh=_mesh(),
      compiler_params=pltpu.CompilerParams(use_tc_tiling_on_sc=False),
      scratch_types=[
          pltpu.VMEM((npt0, K_EDGE), jnp.int32),
          pltpu.VMEM((K_EDGE, 16), jnp.float32),
          pltpu.VMEM_SHARED((n_pad, 16), jnp.float32),
      ],
  )


def _agg_body(npt0, npt1, n_pad, d, nbuf, table, src_i, dst_i, zeros, out,
              *scratch):
  src_v, dst_v = scratch[0], scratch[1]
  rows = list(scratch[2:2 + nbuf])
  sems = list(scratch[2 + nbuf:2 + 2 * nbuf])
  acc = scratch[-1]
  cid = lax.axis_index("c")
  sid = lax.axis_index("s")
  wid = cid * NS + sid
  nloc = jnp.where(cid == 0, npt0, npt1)
  rpt = n_pad // NS
  sl = pl.ds(sid * rpt, rpt)
  pltpu.sync_copy(src_i.at[wid], src_v)
  pltpu.sync_copy(dst_i.at[wid], dst_v)
  pltpu.sync_copy(zeros.at[sl], acc.at[sl])
  plsc.subcore_barrier()

  for c in range(nbuf - 1):
    pltpu.async_copy(table.at[src_v.at[c]], rows[c], sems[c])

  @pl.loop(0, nloc // nbuf)
  def _(i):
    for b in range(nbuf):
      j = nbuf * i + b
      jnext = jnp.minimum(j + nbuf - 1, nloc - 1)
      bn = (b - 1) % nbuf
      pltpu.make_async_copy(table.at[src_v.at[j]], rows[b], sems[b]).wait()
      pltpu.async_copy(table.at[src_v.at[jnext]], rows[bn], sems[bn])
      pltpu.sync_copy(rows[b], acc.at[dst_v.at[j]], add=True)

  for c in range(nbuf - 1):
    pltpu.make_async_copy(table.at[src_v.at[0]], rows[c], sems[c]).wait()
  plsc.subcore_barrier()
  pltpu.sync_copy(acc.at[sl], out.at[cid].at[sl])


def _make_agg(npt0, npt1, n_pad, d, nbuf):
  return pl.kernel(
      functools.partial(_agg_body, npt0, npt1, n_pad, d, nbuf),
      out_type=jax.ShapeDtypeStruct((NC, n_pad, d), jnp.float32),
      mesh=_mesh(),
      compiler_params=pltpu.CompilerParams(use_tc_tiling_on_sc=False),
      scratch_types=[
          pltpu.VMEM((npt0, K_EDGE), jnp.int32),
          pltpu.VMEM((npt0, K_EDGE), jnp.int32),
      ] + [pltpu.VMEM((K_EDGE, d), jnp.float32) for _ in range(nbuf)] + [
          pltpu.SemaphoreType.DMA for _ in range(nbuf)] + [
          pltpu.VMEM_SHARED((n_pad, d), jnp.float32),
      ],
  )


def _s_block(degp, n, r0):
  dsum = degp[0, :, 0:1] + degp[1, :, 0:1]
  s = jnp.where(dsum > 0, lax.rsqrt(jnp.maximum(dsum, 1e-12)), 0.0)
  rows = r0 + lax.broadcasted_iota(jnp.int32, s.shape, 0)
  return jnp.where(rows < n, s, 0.0)


def _lin_first_body(n, r, x_ref, w_ref, degp_ref, o_ref):
  i = pl.program_id(0)
  s = _s_block(degp_ref[...], n, i * r)
  o_ref[...] = s * jnp.dot(x_ref[...], w_ref[...],
                           preferred_element_type=jnp.float32)


def _lin_mid_body(n, r, p_ref, b_ref, w_ref, degp_ref, o_ref):
  i = pl.program_id(0)
  s = _s_block(degp_ref[...], n, i * r)
  z = s * (p_ref[0] + p_ref[1]) + b_ref[...]
  a = jnp.maximum(z, 0.0)
  o_ref[...] = s * jnp.dot(a, w_ref[...], preferred_element_type=jnp.float32)


def _final_body(n, r, nvalid, p_ref, b_ref, degp_ref, o_ref):
  i = pl.program_id(0)
  s = _s_block(degp_ref[...], n, i * r)
  z = s * (p_ref[0] + p_ref[1]) + b_ref[...]
  col = lax.broadcasted_iota(jnp.int32, z.shape, 1)
  valid = col < nvalid
  zm = jnp.where(valid, z, -jnp.inf)
  m = jnp.max(zm, axis=1, keepdims=True)
  e = jnp.where(valid, jnp.exp(zm - m), 0.0)
  lse = jnp.log(jnp.sum(e, axis=1, keepdims=True))
  o_ref[...] = z - m - lse


_R = 512


def _tc_first(n, n_pad, din, dout):
  grid = n_pad // _R
  return pl.pallas_call(
      functools.partial(_lin_first_body, n, _R),
      grid=(grid,),
      in_specs=[
          pl.BlockSpec((_R, din), lambda i: (i, 0)),
          pl.BlockSpec((din, dout), lambda i: (0, 0)),
          pl.BlockSpec((NC, _R, 16), lambda i: (0, i, 0)),
      ],
      out_specs=pl.BlockSpec((_R, dout), lambda i: (i, 0)),
      out_shape=jax.ShapeDtypeStruct((n_pad, dout), jnp.float32),
  )


def _tc_mid(n, n_pad, din, dout):
  grid = n_pad // _R
  return pl.pallas_call(
      functools.partial(_lin_mid_body, n, _R),
      grid=(grid,),
      in_specs=[
          pl.BlockSpec((NC, _R, din), lambda i: (0, i, 0)),
          pl.BlockSpec((1, din), lambda i: (0, 0)),
          pl.BlockSpec((din, dout), lambda i: (0, 0)),
          pl.BlockSpec((NC, _R, 16), lambda i: (0, i, 0)),
      ],
      out_specs=pl.BlockSpec((_R, dout), lambda i: (i, 0)),
      out_shape=jax.ShapeDtypeStruct((n_pad, dout), jnp.float32),
  )


def _tc_final(n, n_pad, d, nvalid):
  grid = n_pad // _R
  return pl.pallas_call(
      functools.partial(_final_body, n, _R, nvalid),
      grid=(grid,),
      in_specs=[
          pl.BlockSpec((NC, _R, d), lambda i: (0, i, 0)),
          pl.BlockSpec((1, d), lambda i: (0, 0)),
          pl.BlockSpec((NC, _R, 16), lambda i: (0, i, 0)),
      ],
      out_specs=pl.BlockSpec((_R, d), lambda i: (i, 0)),
      out_shape=jax.ShapeDtypeStruct((n_pad, d), jnp.float32),
  )


def kernel(x, edge_index, W1, b1, W2, b2, W3, b3):
  n, in_dim = x.shape
  e = edge_index.shape[1]
  h1 = W1.shape[1]
  h2 = W2.shape[1]
  dout = W3.shape[1]
  dout_p = ((dout + 15) // 16) * 16

  n_pad = ((n + _R - 1) // _R) * _R
  e_tot = e + n
  tot = (e_tot + NS * K_EDGE - 1) // (NS * K_EDGE)
  npt0 = min((int(tot * 0.58) + 3) // 4 * 4, 124)
  npt1 = (tot - npt0 + 3) // 4 * 4
  e0 = NS * npt0 * K_EDGE
  e_pad = e0 + NS * npt1 * K_EDGE

  loop = jnp.arange(n, dtype=jnp.int32)
  pad = jnp.full((e_pad - e_tot,), n, dtype=jnp.int32)
  src = jnp.concatenate([edge_index[0].astype(jnp.int32), loop, pad])
  dst = jnp.concatenate([edge_index[1].astype(jnp.int32), loop, pad])

  def _split(a):
    c0 = a[:e0].reshape(NS, npt0, K_EDGE)
    c1 = a[e0:].reshape(NS, npt1, K_EDGE)
    c1 = jnp.pad(c1, ((0, 0), (0, npt0 - npt1), (0, 0)), constant_values=n)
    return jnp.concatenate([c0, c1], axis=0)

  src_i = _split(src)
  dst_i = _split(dst)

  x_pad = jnp.pad(x, ((0, n_pad - n), (0, 0)))
  w3p = jnp.pad(W3, ((0, 0), (0, dout_p - dout)))
  b1r = b1.reshape(1, h1)
  b2r = b2.reshape(1, h2)
  b3r = jnp.pad(b3, (0, dout_p - dout)).reshape(1, dout_p)

  ones16 = jnp.ones((K_EDGE, 16), jnp.float32)
  zeros16 = jnp.zeros((n_pad, 16), jnp.float32)

  degp = _make_deg(npt0, npt1, n_pad)(dst_i, ones16, zeros16)

  t1 = _tc_first(n, n_pad, in_dim, h1)(x_pad, W1, degp)
  t1, sa, da, za = lax.optimization_barrier(
      (t1, src_i, dst_i, jnp.zeros((n_pad, h1), jnp.float32)))
  p1 = _make_agg(npt0, npt1, n_pad, h1, 2)(t1, sa, da, za)
  t2 = _tc_mid(n, n_pad, h1, h2)(p1, b1r, W2, degp)
  t2, sa, da, za = lax.optimization_barrier(
      (t2, src_i, dst_i, jnp.zeros((n_pad, h2), jnp.float32)))
  p2 = _make_agg(npt0, npt1, n_pad, h2, 4)(t2, sa, da, za)
  t3 = _tc_mid(n, n_pad, h2, dout_p)(p2, b2r, w3p, degp)
  t3, sa, da, za = lax.optimization_barrier(
      (t3, src_i, dst_i, jnp.zeros((n_pad, dout_p), jnp.float32)))
  p3 = _make_agg(npt0, npt1, n_pad, dout_p, 4)(t3, sa, da, za)
  o = _tc_final(n, n_pad, dout_p, dout)(p3, b3r, degp)
  return o[:n, :dout]


# speedup vs baseline: 2.4095x; 1.0076x over previous
import functools

import jax
import jax.numpy as jnp
from jax import lax
from jax.experimental import pallas as pl
from jax.experimental.pallas import tpu as pltpu
from jax.experimental.pallas import tpu_sc as plsc

NC = 2
NS = 16
NW = NC * NS
K_EDGE = 96


def _mesh():
  return plsc.VectorSubcoreMesh(
      core_axis_name="c", subcore_axis_name="s", num_cores=NC,
      num_subcores=NS)


def _deg_body(npt0, npt1, n_pad, dst_i, ones, zeros16, out, dst_v, ones_v,
              acc):
  cid = lax.axis_index("c")
  sid = lax.axis_index("s")
  nloc = jnp.where(cid == 0, npt0, npt1)
  start = jnp.where(cid == 0, sid * npt0, NS * npt0 + sid * npt1)
  rpt = n_pad // NS
  sl = pl.ds(sid * rpt, rpt)
  pltpu.sync_copy(dst_i.at[pl.ds(start, npt0)], dst_v)
  pltpu.sync_copy(ones, ones_v)
  pltpu.sync_copy(zeros16.at[sl], acc.at[sl])
  plsc.subcore_barrier()

  @pl.loop(0, nloc)
  def _(j):
    pltpu.sync_copy(ones_v, acc.at[dst_v.at[j]], add=True)

  plsc.subcore_barrier()
  pltpu.sync_copy(acc.at[sl], out.at[cid].at[sl])


def _make_deg(npt0, npt1, n_pad):
  return pl.kernel(
      functools.partial(_deg_body, npt0, npt1, n_pad),
      out_type=jax.ShapeDtypeStruct((NC, n_pad, 16), jnp.float32),
      mesh=_mesh(),
      compiler_params=pltpu.CompilerParams(use_tc_tiling_on_sc=False),
      scratch_types=[
          pltpu.VMEM((npt0, K_EDGE), jnp.int32),
          pltpu.VMEM((K_EDGE, 16), jnp.float32),
          pltpu.VMEM_SHARED((n_pad, 16), jnp.float32),
      ],
  )


def _agg_body(npt0, npt1, n_pad, d, nbuf, table, src_i, dst_i, zeros, out,
              *scratch):
  src_v, dst_v = scratch[0], scratch[1]
  rows = list(scratch[2:2 + nbuf])
  sems = list(scratch[2 + nbuf:2 + 2 * nbuf])
  acc = scratch[-1]
  cid = lax.axis_index("c")
  sid = lax.axis_index("s")
  nloc = jnp.where(cid == 0, npt0, npt1)
  start = jnp.where(cid == 0, sid * npt0, NS * npt0 + sid * npt1)
  rpt = n_pad // NS
  sl = pl.ds(sid * rpt, rpt)
  pltpu.sync_copy(src_i.at[pl.ds(start, npt0)], src_v)
  pltpu.sync_copy(dst_i.at[pl.ds(start, npt0)], dst_v)
  pltpu.sync_copy(zeros.at[sl], acc.at[sl])
  plsc.subcore_barrier()

  for c in range(nbuf - 1):
    pltpu.async_copy(table.at[src_v.at[c]], rows[c], sems[c])

  @pl.loop(0, nloc // nbuf)
  def _(i):
    for b in range(nbuf):
      j = nbuf * i + b
      jnext = jnp.minimum(j + nbuf - 1, nloc - 1)
      bn = (b - 1) % nbuf
      pltpu.make_async_copy(table.at[src_v.at[j]], rows[b], sems[b]).wait()
      pltpu.async_copy(table.at[src_v.at[jnext]], rows[bn], sems[bn])
      pltpu.sync_copy(rows[b], acc.at[dst_v.at[j]], add=True)

  for c in range(nbuf - 1):
    pltpu.make_async_copy(table.at[src_v.at[0]], rows[c], sems[c]).wait()
  plsc.subcore_barrier()
  pltpu.sync_copy(acc.at[sl], out.at[cid].at[sl])


def _make_agg(npt0, npt1, n_pad, d, nbuf):
  return pl.kernel(
      functools.partial(_agg_body, npt0, npt1, n_pad, d, nbuf),
      out_type=jax.ShapeDtypeStruct((NC, n_pad, d), jnp.float32),
      mesh=_mesh(),
      compiler_params=pltpu.CompilerParams(use_tc_tiling_on_sc=False),
      scratch_types=[
          pltpu.VMEM((npt0, K_EDGE), jnp.int32),
          pltpu.VMEM((npt0, K_EDGE), jnp.int32),
      ] + [pltpu.VMEM((K_EDGE, d), jnp.float32) for _ in range(nbuf)] + [
          pltpu.SemaphoreType.DMA for _ in range(nbuf)] + [
          pltpu.VMEM_SHARED((n_pad, d), jnp.float32),
      ],
  )


def _s_block(degp, n, r0):
  dsum = degp[0, :, 0:1] + degp[1, :, 0:1]
  s = jnp.where(dsum > 0, lax.rsqrt(jnp.maximum(dsum, 1e-12)), 0.0)
  rows = r0 + lax.broadcasted_iota(jnp.int32, s.shape, 0)
  return jnp.where(rows < n, s, 0.0)


def _lin_first_body(n, r, x_ref, w_ref, degp_ref, o_ref):
  i = pl.program_id(0)
  s = _s_block(degp_ref[...], n, i * r)
  o_ref[...] = s * jnp.dot(x_ref[...], w_ref[...],
                           preferred_element_type=jnp.float32)


def _lin_mid_body(n, r, p_ref, b_ref, w_ref, degp_ref, o_ref):
  i = pl.program_id(0)
  s = _s_block(degp_ref[...], n, i * r)
  z = s * (p_ref[0] + p_ref[1]) + b_ref[...]
  a = jnp.maximum(z, 0.0)
  o_ref[...] = s * jnp.dot(a, w_ref[...], preferred_element_type=jnp.float32)


def _final_body(n, r, nvalid, p_ref, b_ref, degp_ref, o_ref):
  i = pl.program_id(0)
  s = _s_block(degp_ref[...], n, i * r)
  z = s * (p_ref[0] + p_ref[1]) + b_ref[...]
  col = lax.broadcasted_iota(jnp.int32, z.shape, 1)
  valid = col < nvalid
  zm = jnp.where(valid, z, -jnp.inf)
  m = jnp.max(zm, axis=1, keepdims=True)
  e = jnp.where(valid, jnp.exp(zm - m), 0.0)
  lse = jnp.log(jnp.sum(e, axis=1, keepdims=True))
  o_ref[...] = z - m - lse


_R = 512


def _tc_first(n, n_pad, din, dout):
  grid = n_pad // _R
  return pl.pallas_call(
      functools.partial(_lin_first_body, n, _R),
      grid=(grid,),
      in_specs=[
          pl.BlockSpec((_R, din), lambda i: (i, 0)),
          pl.BlockSpec((din, dout), lambda i: (0, 0)),
          pl.BlockSpec((NC, _R, 16), lambda i: (0, i, 0)),
      ],
      out_specs=pl.BlockSpec((_R, dout), lambda i: (i, 0)),
      out_shape=jax.ShapeDtypeStruct((n_pad, dout), jnp.float32),
  )


def _tc_mid(n, n_pad, din, dout):
  grid = n_pad // _R
  return pl.pallas_call(
      functools.partial(_lin_mid_body, n, _R),
      grid=(grid,),
      in_specs=[
          pl.BlockSpec((NC, _R, din), lambda i: (0, i, 0)),
          pl.BlockSpec((1, din), lambda i: (0, 0)),
          pl.BlockSpec((din, dout), lambda i: (0, 0)),
          pl.BlockSpec((NC, _R, 16), lambda i: (0, i, 0)),
      ],
      out_specs=pl.BlockSpec((_R, dout), lambda i: (i, 0)),
      out_shape=jax.ShapeDtypeStruct((n_pad, dout), jnp.float32),
  )


def _tc_final(n, n_pad, d, nvalid):
  grid = n_pad // _R
  return pl.pallas_call(
      functools.partial(_final_body, n, _R, nvalid),
      grid=(grid,),
      in_specs=[
          pl.BlockSpec((NC, _R, d), lambda i: (0, i, 0)),
          pl.BlockSpec((1, d), lambda i: (0, 0)),
          pl.BlockSpec((NC, _R, 16), lambda i: (0, i, 0)),
      ],
      out_specs=pl.BlockSpec((_R, d), lambda i: (i, 0)),
      out_shape=jax.ShapeDtypeStruct((n_pad, d), jnp.float32),
  )


def kernel(x, edge_index, W1, b1, W2, b2, W3, b3):
  n, in_dim = x.shape
  e = edge_index.shape[1]
  h1 = W1.shape[1]
  h2 = W2.shape[1]
  dout = W3.shape[1]
  dout_p = ((dout + 15) // 16) * 16

  n_pad = ((n + _R - 1) // _R) * _R
  e_tot = e + n
  tot = (e_tot + NS * K_EDGE - 1) // (NS * K_EDGE)
  npt0 = min((int(tot * 0.58) + 3) // 4 * 4, 124)
  npt1 = (tot - npt0 + 3) // 4 * 4
  nrows = NS * (npt0 + npt1) + (npt0 - npt1)
  e_pad = nrows * K_EDGE

  loop = jnp.arange(n, dtype=jnp.int32)
  pad = jnp.full((e_pad - e_tot,), n, dtype=jnp.int32)
  src_i = jnp.concatenate(
      [edge_index[0].astype(jnp.int32), loop, pad]).reshape(nrows, K_EDGE)
  dst_i = jnp.concatenate(
      [edge_index[1].astype(jnp.int32), loop, pad]).reshape(nrows, K_EDGE)

  x_pad = jnp.pad(x, ((0, n_pad - n), (0, 0)))
  w3p = jnp.pad(W3, ((0, 0), (0, dout_p - dout)))
  b1r = b1.reshape(1, h1)
  b2r = b2.reshape(1, h2)
  b3r = jnp.pad(b3, (0, dout_p - dout)).reshape(1, dout_p)

  ones16 = jnp.ones((K_EDGE, 16), jnp.float32)
  zeros16 = jnp.zeros((n_pad, 16), jnp.float32)

  degp = _make_deg(npt0, npt1, n_pad)(dst_i, ones16, zeros16)

  t1 = _tc_first(n, n_pad, in_dim, h1)(x_pad, W1, degp)
  t1, sa, da, za = lax.optimization_barrier(
      (t1, src_i, dst_i, jnp.zeros((n_pad, h1), jnp.float32)))
  p1 = _make_agg(npt0, npt1, n_pad, h1, 2)(t1, sa, da, za)
  t2 = _tc_mid(n, n_pad, h1, h2)(p1, b1r, W2, degp)
  t2, sa, da, za = lax.optimization_barrier(
      (t2, src_i, dst_i, jnp.zeros((n_pad, h2), jnp.float32)))
  p2 = _make_agg(npt0, npt1, n_pad, h2, 4)(t2, sa, da, za)
  t3 = _tc_mid(n, n_pad, h2, dout_p)(p2, b2r, w3p, degp)
  t3, sa, da, za = lax.optimization_barrier(
      (t3, src_i, dst_i, jnp.zeros((n_pad, dout_p), jnp.float32)))
  p3 = _make_agg(npt0, npt1, n_pad, dout_p, 4)(t3, sa, da, za)
  o = _tc_final(n, n_pad, dout_p, dout)(p3, b3r, degp)
  return o[:n, :dout]
